# Initial kernel scaffold; baseline (speedup 1.0000x reference)
import jax
import jax.numpy as jnp
from jax.experimental import pallas as pl


def kernel(node_feats, edge_feats, edge_index, W_proj, b_proj, We1, be1, We2, be2, b_conv, gWih, gWhh, gbih, gbhh, lWih0, lWhh0, lbih0, lbhh0, lWih1, lWhh1, lbih1, lbhh1, lWih2, lWhh2, lbih2, lbhh2, Wd1, bd1, Wd2, bd2):
    raise NotImplementedError("write your pallas kernel here")


# speedup vs baseline: 1.9621x; 1.9621x over previous
import functools

import jax
import jax.numpy as jnp
from jax import lax
from jax.experimental import pallas as pl
from jax.experimental.pallas import tpu as pltpu
from jax.experimental.pallas import tpu_sc as plsc

_NW = 32
_ECH = 128


def _pre_node_body(x_ref, w_ref, b_ref, o_ref):
    o_ref[...] = jnp.maximum(
        jnp.dot(x_ref[...], w_ref[...], preferred_element_type=jnp.float32)
        + b_ref[...],
        0.0,
    )


def _pre_edge_body(x_ref, w_ref, b_ref, o_ref):
    o_ref[...] = jnp.maximum(
        jnp.dot(x_ref[...], w_ref[...], preferred_element_type=jnp.float32)
        + b_ref[...],
        0.0,
    )


def _relu_proj(x, w, b, bm):
    m, k = x.shape
    n = w.shape[1]
    grid = (m // bm,)
    return pl.pallas_call(
        _pre_node_body,
        grid=grid,
        in_specs=[
            pl.BlockSpec((bm, k), lambda i: (i, 0)),
            pl.BlockSpec((k, n), lambda i: (0, 0)),
            pl.BlockSpec((1, n), lambda i: (0, 0)),
        ],
        out_specs=pl.BlockSpec((bm, n), lambda i: (i, 0)),
        out_shape=jax.ShapeDtypeStruct((m, n), jnp.float32),
    )(x, w, b.reshape(1, n))


def _bilinear_body(t_ref, u_ref, w2_ref, b2_ref, r_ref, s_ref, o_ref):
    g = (
        jnp.dot(t_ref[...], w2_ref[...], preferred_element_type=jnp.float32)
        + b2_ref[...]
    )
    urep = jnp.dot(u_ref[...], r_ref[...], preferred_element_type=jnp.float32)
    o_ref[...] = jnp.dot(g * urep, s_ref[...], preferred_element_type=jnp.float32)


def _bilinear(t, u, we2, be2, rmat, smat, be):
    e, eh = t.shape
    hh = we2.shape[1]
    h = u.shape[1]
    grid = (e // be,)
    return pl.pallas_call(
        _bilinear_body,
        grid=grid,
        in_specs=[
            pl.BlockSpec((be, eh), lambda i: (i, 0)),
            pl.BlockSpec((be, h), lambda i: (i, 0)),
            pl.BlockSpec((eh, hh), lambda i: (0, 0)),
            pl.BlockSpec((1, hh), lambda i: (0, 0)),
            pl.BlockSpec((h, hh), lambda i: (0, 0)),
            pl.BlockSpec((hh, h), lambda i: (0, 0)),
        ],
        out_specs=pl.BlockSpec((be, h), lambda i: (i, 0)),
        out_shape=jax.ShapeDtypeStruct((e, h), jnp.float32),
    )(t, u, we2, be2.reshape(1, hh), rmat, smat)


def _gru_body(p_ref, bc_ref, hid_ref, wih_ref, whh_ref, bih_ref, bhh_ref, o_ref):
    h = hid_ref.shape[1]
    agg = p_ref[0] + p_ref[1] + bc_ref[...]
    m = jnp.maximum(agg, 0.0)
    gi = (
        jnp.dot(m, wih_ref[...], preferred_element_type=jnp.float32)
        + bih_ref[...]
    )
    gh = (
        jnp.dot(hid_ref[...], whh_ref[...], preferred_element_type=jnp.float32)
        + bhh_ref[...]
    )
    r = jax.nn.sigmoid(gi[:, :h] + gh[:, :h])
    z = jax.nn.sigmoid(gi[:, h : 2 * h] + gh[:, h : 2 * h])
    n = jnp.tanh(gi[:, 2 * h :] + r * gh[:, 2 * h :])
    o_ref[...] = (1.0 - z) * n + z * hid_ref[...]


def _gru(partials, b_conv, hidden, wih_t, whh_t, bih, bhh, bn):
    nn, h = hidden.shape
    grid = (nn // bn,)
    return pl.pallas_call(
        _gru_body,
        grid=grid,
        in_specs=[
            pl.BlockSpec((2, bn, h), lambda i: (0, i, 0)),
            pl.BlockSpec((1, h), lambda i: (0, 0)),
            pl.BlockSpec((bn, h), lambda i: (i, 0)),
            pl.BlockSpec((h, 3 * h), lambda i: (0, 0)),
            pl.BlockSpec((h, 3 * h), lambda i: (0, 0)),
            pl.BlockSpec((1, 3 * h), lambda i: (0, 0)),
            pl.BlockSpec((1, 3 * h), lambda i: (0, 0)),
        ],
        out_specs=pl.BlockSpec((bn, h), lambda i: (i, 0)),
        out_shape=jax.ShapeDtypeStruct((nn, h), jnp.float32),
    )(
        partials,
        b_conv.reshape(1, h),
        hidden,
        wih_t,
        whh_t,
        bih.reshape(1, 3 * h),
        bhh.reshape(1, 3 * h),
    )


def _s2s_body(
    h_ref,
    wih0_ref, whh0_ref, bih0_ref, bhh0_ref,
    wih1_ref, whh1_ref, bih1_ref, bhh1_ref,
    wih2_ref, whh2_ref, bih2_ref, bhh2_ref,
    wd1_ref, bd1_ref, wd2_ref, bd2_ref,
    o_ref,
):
    hmat = h_ref[...]
    hdim = hmat.shape[1]
    wih = [wih0_ref[...], wih1_ref[...], wih2_ref[...]]
    whh = [whh0_ref[...], whh1_ref[...], whh2_ref[...]]
    bih = [bih0_ref[...], bih1_ref[...], bih2_ref[...]]
    bhh = [bhh0_ref[...], bhh1_ref[...], bhh2_ref[...]]
    q_star = jnp.zeros((1, 2 * hdim), dtype=jnp.float32)
    hs = [jnp.zeros((1, hdim), dtype=jnp.float32) for _ in range(3)]
    cs = [jnp.zeros((1, hdim), dtype=jnp.float32) for _ in range(3)]
    for _ in range(6):
        x = q_star
        for l in range(3):
            g = (
                jnp.dot(x, wih[l], preferred_element_type=jnp.float32)
                + bih[l]
                + jnp.dot(hs[l], whh[l], preferred_element_type=jnp.float32)
                + bhh[l]
            )
            i = jax.nn.sigmoid(g[:, :hdim])
            f = jax.nn.sigmoid(g[:, hdim : 2 * hdim])
            gg = jnp.tanh(g[:, 2 * hdim : 3 * hdim])
            o = jax.nn.sigmoid(g[:, 3 * hdim :])
            cs[l] = f * cs[l] + i * gg
            hs[l] = o * jnp.tanh(cs[l])
            x = hs[l]
        q = x
        e = jnp.dot(hmat, q.reshape(hdim, 1), preferred_element_type=jnp.float32)
        emax = jnp.max(e)
        a = jnp.exp(e - emax)
        denom = jnp.sum(a)
        readout = jnp.sum(a * hmat, axis=0, keepdims=True) / denom
        q_star = jnp.concatenate([q, readout], axis=1)
    out = (
        jnp.dot(
            jnp.maximum(
                jnp.dot(q_star, wd1_ref[...], preferred_element_type=jnp.float32)
                + bd1_ref[...],
                0.0,
            ),
            wd2_ref[...],
            preferred_element_type=jnp.float32,
        )
        + bd2_ref[...]
    )
    o_ref[...] = out


def _set2set_decode(h, lstm_t, wd1, bd1, wd2, bd2):
    nn, hdim = h.shape
    ph = wd1.shape[1]
    tasks = wd2.shape[1]
    args = [h]
    in_specs = [pl.BlockSpec(h.shape, lambda: (0, 0))]
    for (wih_t, whh_t, bih, bhh) in lstm_t:
        for a in (wih_t, whh_t, bih.reshape(1, -1), bhh.reshape(1, -1)):
            args.append(a)
            in_specs.append(pl.BlockSpec(a.shape, lambda: (0, 0)))
    for a in (wd1, bd1.reshape(1, ph), wd2, bd2.reshape(1, tasks)):
        args.append(a)
        in_specs.append(pl.BlockSpec(a.shape, lambda: (0, 0)))
    return pl.pallas_call(
        _s2s_body,
        in_specs=in_specs,
        out_specs=pl.BlockSpec((1, tasks), lambda: (0, 0)),
        out_shape=jax.ShapeDtypeStruct((1, tasks), jnp.float32),
    )(*args)


def _sc_gather(h, src):
    nn, d = h.shape
    etot = src.shape[0]
    nch = etot // _ECH
    per = nch // _NW
    rem = nch % _NW
    mesh = plsc.VectorSubcoreMesh(core_axis_name="c", subcore_axis_name="s")

    @functools.partial(
        pl.kernel,
        mesh=mesh,
        out_type=jax.ShapeDtypeStruct((etot, d), jnp.float32),
        scratch_types=[
            pltpu.VMEM((_ECH,), jnp.int32),
            pltpu.VMEM((_ECH, d), jnp.float32),
            pltpu.SemaphoreType.DMA,
        ],
        compiler_params=pltpu.CompilerParams(use_tc_tiling_on_sc=False),
    )
    def k(h_hbm, src_hbm, out_hbm, idx_v, rows_v, sem):
        c = lax.axis_index("c")
        s = lax.axis_index("s")
        wid = s * 2 + c
        start = wid * per + jnp.minimum(wid, rem)
        cnt = per + jnp.where(wid < rem, 1, 0)

        def body(j, carry):
            off = (start + j) * _ECH
            pltpu.sync_copy(src_hbm.at[pl.ds(off, _ECH)], idx_v)
            pltpu.async_copy(h_hbm.at[idx_v], rows_v, sem).wait()
            pltpu.sync_copy(rows_v, out_hbm.at[pl.ds(off, _ECH)])
            return carry

        lax.fori_loop(0, cnt, body, 0)

    return k(h, src)


def _sc_scatter(msg, dst, zeros_nd):
    etot, d = msg.shape
    nn = zeros_nd.shape[0]
    nch = etot // _ECH
    per = nch // _NW
    rem = nch % _NW
    rpt = nn // 16
    mesh = plsc.VectorSubcoreMesh(core_axis_name="c", subcore_axis_name="s")

    @functools.partial(
        pl.kernel,
        mesh=mesh,
        out_type=jax.ShapeDtypeStruct((2, nn, d), jnp.float32),
        scratch_types=[
            pltpu.VMEM((_ECH,), jnp.int32),
            pltpu.VMEM((_ECH, d), jnp.float32),
            pltpu.VMEM_SHARED((nn, d), jnp.float32),
            pltpu.SemaphoreType.DMA,
        ],
        compiler_params=pltpu.CompilerParams(use_tc_tiling_on_sc=False),
    )
    def k(msg_hbm, dst_hbm, zeros_hbm, out_hbm, idx_v, rows_v, acc_sh, sem):
        c = lax.axis_index("c")
        s = lax.axis_index("s")
        wid = s * 2 + c
        pltpu.sync_copy(
            zeros_hbm.at[pl.ds(s * rpt, rpt)], acc_sh.at[pl.ds(s * rpt, rpt)]
        )
        plsc.subcore_barrier()
        start = wid * per + jnp.minimum(wid, rem)
        cnt = per + jnp.where(wid < rem, 1, 0)

        def body(j, carry):
            off = (start + j) * _ECH
            pltpu.sync_copy(dst_hbm.at[pl.ds(off, _ECH)], idx_v)
            pltpu.sync_copy(msg_hbm.at[pl.ds(off, _ECH)], rows_v)
            pltpu.sync_copy(rows_v, acc_sh.at[idx_v], add=True)
            return carry

        lax.fori_loop(0, cnt, body, 0)
        plsc.subcore_barrier()
        pltpu.sync_copy(
            acc_sh.at[pl.ds(s * rpt, rpt)], out_hbm.at[c, pl.ds(s * rpt, rpt)]
        )

    return k(msg, dst, zeros_nd)


def kernel(node_feats, edge_feats, edge_index, W_proj, b_proj, We1, be1, We2,
           be2, b_conv, gWih, gWhh, gbih, gbhh, lWih0, lWhh0, lbih0, lbhh0,
           lWih1, lWhh1, lbih1, lbhh1, lWih2, lWhh2, lbih2, lbhh2, Wd1, bd1,
           Wd2, bd2):
    nn, _ = node_feats.shape
    etot = edge_feats.shape[0]
    h = W_proj.shape[1]
    hh = We2.shape[1]
    src = edge_index[0]
    dst = edge_index[1]

    cols = jnp.arange(hh, dtype=jnp.int32)
    rmat = (cols[None, :] // h == jnp.arange(h, dtype=jnp.int32)[:, None]).astype(
        jnp.float32
    )
    smat = (cols[:, None] % h == jnp.arange(h, dtype=jnp.int32)[None, :]).astype(
        jnp.float32
    )
    zeros_nd = jnp.zeros((nn, h), dtype=jnp.float32)

    t = _relu_proj(edge_feats, We1, be1, 2000)
    hcur = _relu_proj(node_feats, W_proj, b_proj, 2000)
    hidden = hcur

    wih_t = gWih.T
    whh_t = gWhh.T

    for _ in range(5):
        u = _sc_gather(hcur, src)
        msg = _bilinear(t, u, We2, be2, rmat, smat, 1000)
        partials = _sc_scatter(msg, dst, zeros_nd)
        hidden = _gru(partials, b_conv, hidden, wih_t, whh_t, gbih, gbhh, 2000)
        hcur = hidden

    lstm_t = [
        (lWih0.T, lWhh0.T, lbih0, lbhh0),
        (lWih1.T, lWhh1.T, lbih1, lbhh1),
        (lWih2.T, lWhh2.T, lbih2, lbhh2),
    ]
    return _set2set_decode(hcur, lstm_t, Wd1, bd1, Wd2, bd2)


# speedup vs baseline: 2.1998x; 1.1211x over previous
import functools

import jax
import jax.numpy as jnp
from jax import lax
from jax.experimental import pallas as pl
from jax.experimental.pallas import tpu as pltpu
from jax.experimental.pallas import tpu_sc as plsc

_NW = 32
_ECH = 128
_K = 8


def _proj4_body(x_ref, w_ref, b_ref, o_ref):
    bm = x_ref.shape[0]
    n = w_ref.shape[1]
    del bm
    o_ref[...] = jnp.maximum(
        jnp.dot(x_ref[...], w_ref[...], preferred_element_type=jnp.float32)
        + b_ref[...],
        0.0,
    )


def _relu_proj4(x, w, b, bm):
    m, k = x.shape
    n = w.shape[1]
    grid = (m // bm,)
    return pl.pallas_call(
        _proj4_body,
        grid=grid,
        in_specs=[
            pl.BlockSpec((bm, k), lambda i: (i, 0)),
            pl.BlockSpec((k, n), lambda i: (0, 0)),
            pl.BlockSpec((1, n), lambda i: (0, 0)),
        ],
        out_specs=pl.BlockSpec((bm, n), lambda i: (i, 0)),
        out_shape=jax.ShapeDtypeStruct((m, n), jnp.float32),
    )(x, w, b.reshape(1, n))


def _pre_edge_body(x_ref, w_ref, b_ref, o_ref):
    o_ref[...] = jnp.maximum(
        jnp.dot(x_ref[...], w_ref[...], preferred_element_type=jnp.float32)
        + b_ref[...],
        0.0,
    ).astype(jnp.bfloat16)


def _pre_edge(x, w, b, bm):
    m, k = x.shape
    n = w.shape[1]
    grid = (m // bm,)
    return pl.pallas_call(
        _pre_edge_body,
        grid=grid,
        in_specs=[
            pl.BlockSpec((bm, k), lambda i: (i, 0)),
            pl.BlockSpec((k, n), lambda i: (0, 0)),
            pl.BlockSpec((1, n), lambda i: (0, 0)),
        ],
        out_specs=pl.BlockSpec((bm, n), lambda i: (i, 0)),
        out_shape=jax.ShapeDtypeStruct((m, n), jnp.bfloat16),
    )(x, w, b.reshape(1, n))


def _bilinear_body(t_ref, u_ref, w2_ref, b2_ref, r_ref, s_ref, o_ref):
    u = u_ref[...].astype(jnp.bfloat16)
    g = (
        jnp.dot(t_ref[...], w2_ref[...], preferred_element_type=jnp.float32)
        + b2_ref[...]
    )
    urep = jnp.dot(u, r_ref[...], preferred_element_type=jnp.float32)
    prod = (g * urep).astype(jnp.bfloat16)
    o_ref[...] = jnp.dot(prod, s_ref[...], preferred_element_type=jnp.float32)


def _bilinear(t, u, we2, be2, rmat, smat, be):
    e, eh = t.shape
    hh = we2.shape[1]
    h = rmat.shape[0]
    grid = (e // be,)
    return pl.pallas_call(
        _bilinear_body,
        grid=grid,
        in_specs=[
            pl.BlockSpec((be, eh), lambda i: (i, 0)),
            pl.BlockSpec((be, h), lambda i: (i, 0)),
            pl.BlockSpec((eh, hh), lambda i: (0, 0)),
            pl.BlockSpec((1, hh), lambda i: (0, 0)),
            pl.BlockSpec((h, hh), lambda i: (0, 0)),
            pl.BlockSpec((hh, h), lambda i: (0, 0)),
        ],
        out_specs=pl.BlockSpec((be, h), lambda i: (i, 0)),
        out_shape=jax.ShapeDtypeStruct((e, h), jnp.float32),
    )(t, u, we2, be2.reshape(1, hh), rmat, smat)


def _gru_body(p0_ref, p1_ref, bc_ref, hid_ref, wih_ref, whh_ref, bih_ref,
              bhh_ref, o_ref):
    h = bc_ref.shape[1]
    agg = p0_ref[...] + p1_ref[...] + bc_ref[...]
    hid = hid_ref[...]
    m = jnp.maximum(agg, 0.0)
    gi = (
        jnp.dot(m, wih_ref[...], preferred_element_type=jnp.float32)
        + bih_ref[...]
    )
    gh = (
        jnp.dot(hid, whh_ref[...], preferred_element_type=jnp.float32)
        + bhh_ref[...]
    )
    r = jax.nn.sigmoid(gi[:, :h] + gh[:, :h])
    z = jax.nn.sigmoid(gi[:, h : 2 * h] + gh[:, h : 2 * h])
    n = jnp.tanh(gi[:, 2 * h :] + r * gh[:, 2 * h :])
    o_ref[...] = (1.0 - z) * n + z * hid


def _gru(p0, p1, b_conv, hid, wih_t, whh_t, bih, bhh, bn):
    h = b_conv.shape[0]
    nn = hid.shape[0]
    grid = (nn // bn,)
    return pl.pallas_call(
        _gru_body,
        grid=grid,
        in_specs=[
            pl.BlockSpec((bn, h), lambda i: (i, 0)),
            pl.BlockSpec((bn, h), lambda i: (i, 0)),
            pl.BlockSpec((1, h), lambda i: (0, 0)),
            pl.BlockSpec((bn, h), lambda i: (i, 0)),
            pl.BlockSpec((h, 3 * h), lambda i: (0, 0)),
            pl.BlockSpec((h, 3 * h), lambda i: (0, 0)),
            pl.BlockSpec((1, 3 * h), lambda i: (0, 0)),
            pl.BlockSpec((1, 3 * h), lambda i: (0, 0)),
        ],
        out_specs=pl.BlockSpec((bn, h), lambda i: (i, 0)),
        out_shape=jax.ShapeDtypeStruct((nn, h), jnp.float32),
    )(
        p0,
        p1,
        b_conv.reshape(1, h),
        hid,
        wih_t,
        whh_t,
        bih.reshape(1, 3 * h),
        bhh.reshape(1, 3 * h),
    )


def _s2s_body(
    h_ref,
    wih0_ref, whh0_ref, bih0_ref, bhh0_ref,
    wih1_ref, whh1_ref, bih1_ref, bhh1_ref,
    wih2_ref, whh2_ref, bih2_ref, bhh2_ref,
    wd1_ref, bd1_ref, wd2_ref, bd2_ref,
    o_ref,
):
    hdim = h_ref.shape[1]
    hmat = h_ref[...]
    wih = [wih0_ref[...], wih1_ref[...], wih2_ref[...]]
    whh = [whh0_ref[...], whh1_ref[...], whh2_ref[...]]
    bih = [bih0_ref[...], bih1_ref[...], bih2_ref[...]]
    bhh = [bhh0_ref[...], bhh1_ref[...], bhh2_ref[...]]
    q_star = jnp.zeros((1, 2 * hdim), dtype=jnp.float32)
    hs = [jnp.zeros((1, hdim), dtype=jnp.float32) for _ in range(3)]
    cs = [jnp.zeros((1, hdim), dtype=jnp.float32) for _ in range(3)]
    for _ in range(6):
        x = q_star
        for l in range(3):
            g = (
                jnp.dot(x, wih[l], preferred_element_type=jnp.float32)
                + bih[l]
                + jnp.dot(hs[l], whh[l], preferred_element_type=jnp.float32)
                + bhh[l]
            )
            i = jax.nn.sigmoid(g[:, :hdim])
            f = jax.nn.sigmoid(g[:, hdim : 2 * hdim])
            gg = jnp.tanh(g[:, 2 * hdim : 3 * hdim])
            o = jax.nn.sigmoid(g[:, 3 * hdim :])
            cs[l] = f * cs[l] + i * gg
            hs[l] = o * jnp.tanh(cs[l])
            x = hs[l]
        q = x
        e = jnp.dot(hmat, q.reshape(hdim, 1), preferred_element_type=jnp.float32)
        emax = jnp.max(e)
        a = jnp.exp(e - emax)
        denom = jnp.sum(a)
        readout = jnp.sum(a * hmat, axis=0, keepdims=True) / denom
        q_star = jnp.concatenate([q, readout], axis=1)
    out = (
        jnp.dot(
            jnp.maximum(
                jnp.dot(q_star, wd1_ref[...], preferred_element_type=jnp.float32)
                + bd1_ref[...],
                0.0,
            ),
            wd2_ref[...],
            preferred_element_type=jnp.float32,
        )
        + bd2_ref[...]
    )
    o_ref[...] = out


def _set2set_decode(hmat, lstm_t, wd1, bd1, wd2, bd2):
    ph = wd1.shape[1]
    tasks = wd2.shape[1]
    args = [hmat]
    in_specs = [pl.BlockSpec(hmat.shape, lambda: (0, 0))]
    for (wih_t, whh_t, bih, bhh) in lstm_t:
        for a in (wih_t, whh_t, bih.reshape(1, -1), bhh.reshape(1, -1)):
            args.append(a)
            in_specs.append(pl.BlockSpec(a.shape, lambda: (0, 0)))
    for a in (wd1, bd1.reshape(1, ph), wd2, bd2.reshape(1, tasks)):
        args.append(a)
        in_specs.append(pl.BlockSpec(a.shape, lambda: (0, 0)))
    return pl.pallas_call(
        _s2s_body,
        in_specs=in_specs,
        out_specs=pl.BlockSpec((1, tasks), lambda: (0, 0)),
        out_shape=jax.ShapeDtypeStruct((1, tasks), jnp.float32),
    )(*args)


def _sc_gather(h, srcm):
    nn, d = h.shape
    nch = srcm.shape[0]
    etot = nch * _ECH
    per = nch // _NW
    rem = nch % _NW
    mesh = plsc.VectorSubcoreMesh(core_axis_name="c", subcore_axis_name="s")

    @functools.partial(
        pl.kernel,
        mesh=mesh,
        out_type=jax.ShapeDtypeStruct((etot, d), jnp.float32),
        scratch_types=[
            pltpu.VMEM((_K, _ECH), jnp.int32),
            pltpu.VMEM((_K * _ECH, d), jnp.float32),
            pltpu.SemaphoreType.DMA,
        ],
        compiler_params=pltpu.CompilerParams(use_tc_tiling_on_sc=False),
    )
    def k(h_hbm, srcm_hbm, out_hbm, idx2, rows_v, sem):
        c = lax.axis_index("c")
        s = lax.axis_index("s")
        wid = s * 2 + c
        base_row = wid * per

        def outer(row0, nrows):
            pltpu.sync_copy(srcm_hbm.at[pl.ds(row0, nrows)], idx2.at[pl.ds(0, nrows)])
            handles = []
            for j in range(nrows):
                handles.append(
                    pltpu.async_copy(
                        h_hbm.at[idx2.at[j]],
                        rows_v.at[pl.ds(j * _ECH, _ECH)],
                        sem,
                    )
                )
            for hd in handles:
                hd.wait()
            pltpu.sync_copy(
                rows_v.at[pl.ds(0, nrows * _ECH)],
                out_hbm.at[pl.ds(row0 * _ECH, nrows * _ECH)],
            )

        nfull, tail = divmod(per, _K)
        for o in range(nfull):
            outer(base_row + o * _K, _K)
        if tail:
            outer(base_row + nfull * _K, tail)
        if rem:
            @pl.when(wid < rem)
            def _():
                outer(_NW * per + wid, 1)

    return k(h, srcm)


def _sc_scatter(msg, dstm, zeros_nd):
    nch = dstm.shape[0]
    nn, d = zeros_nd.shape
    per = nch // _NW
    rem = nch % _NW
    rpt = nn // 16
    mesh = plsc.VectorSubcoreMesh(core_axis_name="c", subcore_axis_name="s")

    @functools.partial(
        pl.kernel,
        mesh=mesh,
        out_type=jax.ShapeDtypeStruct((2, nn, d), jnp.float32),
        scratch_types=[
            pltpu.VMEM((_K, _ECH), jnp.int32),
            pltpu.VMEM((_K * _ECH, d), jnp.float32),
            pltpu.VMEM_SHARED((nn, d), jnp.float32),
            pltpu.SemaphoreType.DMA,
        ],
        compiler_params=pltpu.CompilerParams(use_tc_tiling_on_sc=False),
    )
    def k(msg_hbm, dstm_hbm, zeros_hbm, out_hbm, idx2, rows_v, acc_sh, sem):
        c = lax.axis_index("c")
        s = lax.axis_index("s")
        wid = s * 2 + c
        base_row = wid * per
        msg_r = msg_hbm
        pltpu.sync_copy(
            zeros_hbm.at[pl.ds(s * rpt, rpt)], acc_sh.at[pl.ds(s * rpt, rpt)]
        )
        plsc.subcore_barrier()

        def outer(row0, nrows):
            pltpu.sync_copy(dstm_hbm.at[pl.ds(row0, nrows)], idx2.at[pl.ds(0, nrows)])
            pltpu.sync_copy(
                msg_r.at[pl.ds(row0 * _ECH, nrows * _ECH)],
                rows_v.at[pl.ds(0, nrows * _ECH)],
            )
            handles = []
            for j in range(nrows):
                handles.append(
                    pltpu.async_copy(
                        rows_v.at[pl.ds(j * _ECH, _ECH)],
                        acc_sh.at[idx2.at[j]],
                        sem,
                        add=True,
                    )
                )
            for hd in handles:
                hd.wait()

        nfull, tail = divmod(per, _K)
        for o in range(nfull):
            outer(base_row + o * _K, _K)
        if tail:
            outer(base_row + nfull * _K, tail)
        if rem:
            @pl.when(wid < rem)
            def _():
                outer(_NW * per + wid, 1)

        plsc.subcore_barrier()
        pltpu.sync_copy(
            acc_sh.at[pl.ds(s * rpt, rpt)], out_hbm.at[c, pl.ds(s * rpt, rpt)]
        )

    return k(msg, dstm, zeros_nd)


def kernel(node_feats, edge_feats, edge_index, W_proj, b_proj, We1, be1, We2,
           be2, b_conv, gWih, gWhh, gbih, gbhh, lWih0, lWhh0, lbih0, lbhh0,
           lWih1, lWhh1, lbih1, lbhh1, lWih2, lWhh2, lbih2, lbhh2, Wd1, bd1,
           Wd2, bd2):
    nn = node_feats.shape[0]
    etot = edge_feats.shape[0]
    h = W_proj.shape[1]
    hh = We2.shape[1]
    srcm = edge_index[0].reshape(etot // _ECH, _ECH)
    dstm = edge_index[1].reshape(etot // _ECH, _ECH)

    cols = jnp.arange(hh, dtype=jnp.int32)
    rmat = (cols[None, :] // h == jnp.arange(h, dtype=jnp.int32)[:, None]).astype(
        jnp.bfloat16
    )
    smat = (cols[:, None] % h == jnp.arange(h, dtype=jnp.int32)[None, :]).astype(
        jnp.bfloat16
    )
    zeros_nd = jnp.zeros((nn, h), dtype=jnp.float32)
    we2_b = We2.astype(jnp.bfloat16)

    t = _pre_edge(edge_feats, We1, be1, 2000)
    hcur = _relu_proj4(node_feats, W_proj, b_proj, 10000)
    hidden = hcur

    wih_t = gWih.T
    whh_t = gWhh.T

    for _ in range(5):
        u = _sc_gather(hcur, srcm)
        msg = _bilinear(t, u, we2_b, be2, rmat, smat, 1600)
        partials = _sc_scatter(msg, dstm, zeros_nd)
        hidden = _gru(partials[0], partials[1], b_conv, hidden, wih_t, whh_t,
                      gbih, gbhh, 10000)
        hcur = hidden

    lstm_t = [
        (lWih0.T, lWhh0.T, lbih0, lbhh0),
        (lWih1.T, lWhh1.T, lbih1, lbhh1),
        (lWih2.T, lWhh2.T, lbih2, lbhh2),
    ]
    return _set2set_decode(hcur, lstm_t, Wd1, bd1, Wd2, bd2)


# speedup vs baseline: 2.3269x; 1.0578x over previous
import functools

import jax
import jax.numpy as jnp
from jax import lax
from jax.experimental import pallas as pl
from jax.experimental.pallas import tpu as pltpu
from jax.experimental.pallas import tpu_sc as plsc

_NW = 32
_ECH = 128
_K = 8


def _proj4_body(x_ref, w_ref, b_ref, o_ref):
    bm = x_ref.shape[0]
    n = w_ref.shape[1]
    del bm
    o_ref[...] = jnp.maximum(
        jnp.dot(x_ref[...], w_ref[...], preferred_element_type=jnp.float32)
        + b_ref[...],
        0.0,
    )


def _relu_proj4(x, w, b, bm):
    m, k = x.shape
    n = w.shape[1]
    grid = (m // bm,)
    return pl.pallas_call(
        _proj4_body,
        grid=grid,
        in_specs=[
            pl.BlockSpec((bm, k), lambda i: (i, 0)),
            pl.BlockSpec((k, n), lambda i: (0, 0)),
            pl.BlockSpec((1, n), lambda i: (0, 0)),
        ],
        out_specs=pl.BlockSpec((bm, n), lambda i: (i, 0)),
        out_shape=jax.ShapeDtypeStruct((m, n), jnp.float32),
    )(x, w, b.reshape(1, n))


def _pre_edge_body(x_ref, w_ref, b_ref, o_ref):
    o_ref[...] = jnp.maximum(
        jnp.dot(x_ref[...], w_ref[...], preferred_element_type=jnp.float32)
        + b_ref[...],
        0.0,
    ).astype(jnp.bfloat16)


def _pre_edge(x, w, b, bm):
    m, k = x.shape
    n = w.shape[1]
    grid = (m // bm,)
    return pl.pallas_call(
        _pre_edge_body,
        grid=grid,
        in_specs=[
            pl.BlockSpec((bm, k), lambda i: (i, 0)),
            pl.BlockSpec((k, n), lambda i: (0, 0)),
            pl.BlockSpec((1, n), lambda i: (0, 0)),
        ],
        out_specs=pl.BlockSpec((bm, n), lambda i: (i, 0)),
        out_shape=jax.ShapeDtypeStruct((m, n), jnp.bfloat16),
    )(x, w, b.reshape(1, n))


def _bilinear_body(t_ref, u_ref, w2_ref, b2r_ref, r_ref, s_ref, o_ref):
    u = u_ref[...].astype(jnp.bfloat16)
    g = jnp.dot(t_ref[...], w2_ref[...], preferred_element_type=jnp.float32)
    urep = jnp.dot(u, r_ref[...], preferred_element_type=jnp.float32)
    prod = (g * urep).astype(jnp.bfloat16)
    o_ref[...] = jnp.dot(
        prod, s_ref[...], preferred_element_type=jnp.float32
    ) + jnp.dot(u, b2r_ref[...], preferred_element_type=jnp.float32)


def _bilinear(t, u, we2, b2r, rmat, smat, be):
    e, eh = t.shape
    hh = we2.shape[1]
    h = rmat.shape[0]
    grid = (e // be,)
    return pl.pallas_call(
        _bilinear_body,
        grid=grid,
        in_specs=[
            pl.BlockSpec((be, eh), lambda i: (i, 0)),
            pl.BlockSpec((be, h), lambda i: (i, 0)),
            pl.BlockSpec((eh, hh), lambda i: (0, 0)),
            pl.BlockSpec((h, h), lambda i: (0, 0)),
            pl.BlockSpec((h, hh), lambda i: (0, 0)),
            pl.BlockSpec((hh, h), lambda i: (0, 0)),
        ],
        out_specs=pl.BlockSpec((be, h), lambda i: (i, 0)),
        out_shape=jax.ShapeDtypeStruct((e, h), jnp.float32),
    )(t, u, we2, b2r, rmat, smat)


def _gru_body(p0_ref, p1_ref, bc_ref, hid_ref, wih_ref, whh_ref, bih_ref,
              bhh_ref, o_ref):
    h = bc_ref.shape[1]
    agg = p0_ref[...] + p1_ref[...] + bc_ref[...]
    hid = hid_ref[...]
    m = jnp.maximum(agg, 0.0)
    gi = (
        jnp.dot(m, wih_ref[...], preferred_element_type=jnp.float32)
        + bih_ref[...]
    )
    gh = (
        jnp.dot(hid, whh_ref[...], preferred_element_type=jnp.float32)
        + bhh_ref[...]
    )
    r = jax.nn.sigmoid(gi[:, :h] + gh[:, :h])
    z = jax.nn.sigmoid(gi[:, h : 2 * h] + gh[:, h : 2 * h])
    n = jnp.tanh(gi[:, 2 * h :] + r * gh[:, 2 * h :])
    o_ref[...] = (1.0 - z) * n + z * hid


def _gru(p0, p1, b_conv, hid, wih_t, whh_t, bih, bhh, bn):
    h = b_conv.shape[0]
    nn = hid.shape[0]
    grid = (nn // bn,)
    return pl.pallas_call(
        _gru_body,
        grid=grid,
        in_specs=[
            pl.BlockSpec((bn, h), lambda i: (i, 0)),
            pl.BlockSpec((bn, h), lambda i: (i, 0)),
            pl.BlockSpec((1, h), lambda i: (0, 0)),
            pl.BlockSpec((bn, h), lambda i: (i, 0)),
            pl.BlockSpec((h, 3 * h), lambda i: (0, 0)),
            pl.BlockSpec((h, 3 * h), lambda i: (0, 0)),
            pl.BlockSpec((1, 3 * h), lambda i: (0, 0)),
            pl.BlockSpec((1, 3 * h), lambda i: (0, 0)),
        ],
        out_specs=pl.BlockSpec((bn, h), lambda i: (i, 0)),
        out_shape=jax.ShapeDtypeStruct((nn, h), jnp.float32),
    )(
        p0,
        p1,
        b_conv.reshape(1, h),
        hid,
        wih_t,
        whh_t,
        bih.reshape(1, 3 * h),
        bhh.reshape(1, 3 * h),
    )


def _s2s_body(
    h_ref,
    wih0_ref, whh0_ref, bih0_ref, bhh0_ref,
    wih1_ref, whh1_ref, bih1_ref, bhh1_ref,
    wih2_ref, whh2_ref, bih2_ref, bhh2_ref,
    wd1_ref, bd1_ref, wd2_ref, bd2_ref,
    o_ref,
):
    hdim = h_ref.shape[1]
    hmat = h_ref[...]
    wih = [wih0_ref[...], wih1_ref[...], wih2_ref[...]]
    whh = [whh0_ref[...], whh1_ref[...], whh2_ref[...]]
    bih = [bih0_ref[...], bih1_ref[...], bih2_ref[...]]
    bhh = [bhh0_ref[...], bhh1_ref[...], bhh2_ref[...]]
    q_star = jnp.zeros((1, 2 * hdim), dtype=jnp.float32)
    hs = [jnp.zeros((1, hdim), dtype=jnp.float32) for _ in range(3)]
    cs = [jnp.zeros((1, hdim), dtype=jnp.float32) for _ in range(3)]
    for _ in range(6):
        x = q_star
        for l in range(3):
            g = (
                jnp.dot(x, wih[l], preferred_element_type=jnp.float32)
                + bih[l]
                + jnp.dot(hs[l], whh[l], preferred_element_type=jnp.float32)
                + bhh[l]
            )
            i = jax.nn.sigmoid(g[:, :hdim])
            f = jax.nn.sigmoid(g[:, hdim : 2 * hdim])
            gg = jnp.tanh(g[:, 2 * hdim : 3 * hdim])
            o = jax.nn.sigmoid(g[:, 3 * hdim :])
            cs[l] = f * cs[l] + i * gg
            hs[l] = o * jnp.tanh(cs[l])
            x = hs[l]
        q = x
        e = jnp.dot(hmat, q.reshape(hdim, 1), preferred_element_type=jnp.float32)
        emax = jnp.max(e)
        a = jnp.exp(e - emax)
        denom = jnp.sum(a)
        readout = jnp.sum(a * hmat, axis=0, keepdims=True) / denom
        q_star = jnp.concatenate([q, readout], axis=1)
    out = (
        jnp.dot(
            jnp.maximum(
                jnp.dot(q_star, wd1_ref[...], preferred_element_type=jnp.float32)
                + bd1_ref[...],
                0.0,
            ),
            wd2_ref[...],
            preferred_element_type=jnp.float32,
        )
        + bd2_ref[...]
    )
    o_ref[...] = out


def _set2set_decode(hmat, lstm_t, wd1, bd1, wd2, bd2):
    ph = wd1.shape[1]
    tasks = wd2.shape[1]
    args = [hmat]
    in_specs = [pl.BlockSpec(hmat.shape, lambda: (0, 0))]
    for (wih_t, whh_t, bih, bhh) in lstm_t:
        for a in (wih_t, whh_t, bih.reshape(1, -1), bhh.reshape(1, -1)):
            args.append(a)
            in_specs.append(pl.BlockSpec(a.shape, lambda: (0, 0)))
    for a in (wd1, bd1.reshape(1, ph), wd2, bd2.reshape(1, tasks)):
        args.append(a)
        in_specs.append(pl.BlockSpec(a.shape, lambda: (0, 0)))
    return pl.pallas_call(
        _s2s_body,
        in_specs=in_specs,
        out_specs=pl.BlockSpec((1, tasks), lambda: (0, 0)),
        out_shape=jax.ShapeDtypeStruct((1, tasks), jnp.float32),
    )(*args)


def _sc_gather(h, srcm):
    nn, d = h.shape
    nch = srcm.shape[0]
    etot = nch * _ECH
    per = nch // _NW
    rem = nch % _NW
    mesh = plsc.VectorSubcoreMesh(core_axis_name="c", subcore_axis_name="s")

    @functools.partial(
        pl.kernel,
        mesh=mesh,
        out_type=jax.ShapeDtypeStruct((etot, d), jnp.float32),
        scratch_types=[
            pltpu.VMEM((_K, _ECH), jnp.int32),
            pltpu.VMEM((_K * _ECH, d), jnp.float32),
            pltpu.SemaphoreType.DMA,
        ],
        compiler_params=pltpu.CompilerParams(use_tc_tiling_on_sc=False),
    )
    def k(h_hbm, srcm_hbm, out_hbm, idx2, rows_v, sem):
        c = lax.axis_index("c")
        s = lax.axis_index("s")
        wid = s * 2 + c
        base_row = wid * per

        def outer(row0, nrows):
            pltpu.sync_copy(srcm_hbm.at[pl.ds(row0, nrows)], idx2.at[pl.ds(0, nrows)])
            handles = []
            for j in range(nrows):
                handles.append(
                    pltpu.async_copy(
                        h_hbm.at[idx2.at[j]],
                        rows_v.at[pl.ds(j * _ECH, _ECH)],
                        sem,
                    )
                )
            for hd in handles:
                hd.wait()
            pltpu.sync_copy(
                rows_v.at[pl.ds(0, nrows * _ECH)],
                out_hbm.at[pl.ds(row0 * _ECH, nrows * _ECH)],
            )

        nfull, tail = divmod(per, _K)
        for o in range(nfull):
            outer(base_row + o * _K, _K)
        if tail:
            outer(base_row + nfull * _K, tail)
        if rem:
            @pl.when(wid < rem)
            def _():
                outer(_NW * per + wid, 1)

    return k(h, srcm)


def _sc_scatter(msg, dstm, zeros_nd):
    nch = dstm.shape[0]
    nn, d = zeros_nd.shape
    per = nch // _NW
    rem = nch % _NW
    rpt = nn // 16
    mesh = plsc.VectorSubcoreMesh(core_axis_name="c", subcore_axis_name="s")

    @functools.partial(
        pl.kernel,
        mesh=mesh,
        out_type=jax.ShapeDtypeStruct((2, nn, d), jnp.float32),
        scratch_types=[
            pltpu.VMEM((_K, _ECH), jnp.int32),
            pltpu.VMEM((_K * _ECH, d), jnp.float32),
            pltpu.VMEM_SHARED((nn, d), jnp.float32),
            pltpu.SemaphoreType.DMA,
        ],
        compiler_params=pltpu.CompilerParams(use_tc_tiling_on_sc=False),
    )
    def k(msg_hbm, dstm_hbm, zeros_hbm, out_hbm, idx2, rows_v, acc_sh, sem):
        c = lax.axis_index("c")
        s = lax.axis_index("s")
        wid = s * 2 + c
        base_row = wid * per
        msg_r = msg_hbm
        pltpu.sync_copy(
            zeros_hbm.at[pl.ds(s * rpt, rpt)], acc_sh.at[pl.ds(s * rpt, rpt)]
        )
        plsc.subcore_barrier()

        def outer(row0, nrows):
            pltpu.sync_copy(dstm_hbm.at[pl.ds(row0, nrows)], idx2.at[pl.ds(0, nrows)])
            pltpu.sync_copy(
                msg_r.at[pl.ds(row0 * _ECH, nrows * _ECH)],
                rows_v.at[pl.ds(0, nrows * _ECH)],
            )
            handles = []
            for j in range(nrows):
                handles.append(
                    pltpu.async_copy(
                        rows_v.at[pl.ds(j * _ECH, _ECH)],
                        acc_sh.at[idx2.at[j]],
                        sem,
                        add=True,
                    )
                )
            for hd in handles:
                hd.wait()

        nfull, tail = divmod(per, _K)
        for o in range(nfull):
            outer(base_row + o * _K, _K)
        if tail:
            outer(base_row + nfull * _K, tail)
        if rem:
            @pl.when(wid < rem)
            def _():
                outer(_NW * per + wid, 1)

        plsc.subcore_barrier()
        pltpu.sync_copy(
            acc_sh.at[pl.ds(s * rpt, rpt)], out_hbm.at[c, pl.ds(s * rpt, rpt)]
        )

    return k(msg, dstm, zeros_nd)


def kernel(node_feats, edge_feats, edge_index, W_proj, b_proj, We1, be1, We2,
           be2, b_conv, gWih, gWhh, gbih, gbhh, lWih0, lWhh0, lbih0, lbhh0,
           lWih1, lWhh1, lbih1, lbhh1, lWih2, lWhh2, lbih2, lbhh2, Wd1, bd1,
           Wd2, bd2):
    nn = node_feats.shape[0]
    etot = edge_feats.shape[0]
    h = W_proj.shape[1]
    hh = We2.shape[1]
    srcm = edge_index[0].reshape(etot // _ECH, _ECH)
    dstm = edge_index[1].reshape(etot // _ECH, _ECH)

    cols = jnp.arange(hh, dtype=jnp.int32)
    rmat = (cols[None, :] // h == jnp.arange(h, dtype=jnp.int32)[:, None]).astype(
        jnp.bfloat16
    )
    smat = (cols[:, None] % h == jnp.arange(h, dtype=jnp.int32)[None, :]).astype(
        jnp.bfloat16
    )
    zeros_nd = jnp.zeros((nn, h), dtype=jnp.float32)
    we2_b = We2.astype(jnp.bfloat16)
    b2r = be2.reshape(h, h).astype(jnp.bfloat16)

    t = _pre_edge(edge_feats, We1, be1, 2000)
    hcur = _relu_proj4(node_feats, W_proj, b_proj, 10000)
    hidden = hcur

    wih_t = gWih.T
    whh_t = gWhh.T

    for _ in range(5):
        u = _sc_gather(hcur, srcm)
        msg = _bilinear(t, u, we2_b, b2r, rmat, smat, 1600)
        partials = _sc_scatter(msg, dstm, zeros_nd)
        hidden = _gru(partials[0], partials[1], b_conv, hidden, wih_t, whh_t,
                      gbih, gbhh, 10000)
        hcur = hidden

    lstm_t = [
        (lWih0.T, lWhh0.T, lbih0, lbhh0),
        (lWih1.T, lWhh1.T, lbih1, lbhh1),
        (lWih2.T, lWhh2.T, lbih2, lbhh2),
    ]
    return _set2set_decode(hcur, lstm_t, Wd1, bd1, Wd2, bd2)
